# Initial kernel scaffold; baseline (speedup 1.0000x reference)
import functools

import jax
import jax.numpy as jnp
from jax import lax
from jax.experimental import pallas as pl
from jax.experimental.pallas import tpu as pltpu
from jax.experimental.pallas import tpu_sc as plsc

B = 16384
D_MOVIE = 64
G = 8
D_GENRE = 16
D_COLL = 16
D_OV = 768
HID = 64
RANK = 64

NC = 2
NS = 16
NW = NC * NS
S = B // NW
C = 128
NCH = S // C
OVC = 32
NOV = S // OVC


def _sc_body(x4_hbm, x8_hbm, movie_hbm, meta_hbm, ov_hbm, gtab_hbm, ctab_hbm,
             movie_out, gsum_out, coll_out, ov_out,
             idx4_v, idx8_v, col_v, meta_v, grow_v, gacc_v, crow_v, mrow_v,
             ovb0, ovb1, sem_s, sem_i, sem_o):
    wid = lax.axis_index("s") * NC + lax.axis_index("c")
    base = wid * S

    pltpu.sync_copy(x4_hbm.at[wid], idx4_v)
    pltpu.sync_copy(x8_hbm.at[wid], idx8_v)

    ovb = [ovb0, ovb1]
    cp_in = pltpu.async_copy(ov_hbm.at[idx8_v.at[0]], ovb[0], sem_i)

    iota = lax.iota(jnp.int32, 16)
    for j in range(NCH):
        ids = idx4_v.at[j]
        rbase = base + j * C
        cp_m = pltpu.async_copy(movie_hbm.at[ids], mrow_v, sem_s)
        cp_meta = pltpu.async_copy(meta_hbm.at[ids], meta_v, sem_s)
        cp_meta.wait()
        for k in range(G + 1):
            kk = jnp.full((16,), k, jnp.int32)
            for i in range(C // 16):
                rows = iota + (i * 16)
                vals = plsc.load_gather(meta_v, [rows, kk])
                col_v[k, pl.ds(i * 16, 16)] = vals
        cps = [pltpu.async_copy(gtab_hbm.at[col_v.at[k]], grow_v.at[k], sem_s)
               for k in range(G)]
        cp_c = pltpu.async_copy(ctab_hbm.at[col_v.at[G]], crow_v, sem_s)
        for cp in cps:
            cp.wait()
        for r in range(C):
            acc = grow_v[0, r, :]
            for k in range(1, G):
                acc = acc + grow_v[k, r, :]
            gacc_v[r, :] = acc
        pltpu.sync_copy(gacc_v, gsum_out.at[pl.ds(rbase, C)])
        cp_c.wait()
        pltpu.sync_copy(crow_v, coll_out.at[pl.ds(rbase, C)])
        cp_m.wait()
        pltpu.sync_copy(mrow_v, movie_out.at[pl.ds(rbase, C)])

    cp_out = [None] * NOV
    for j in range(NOV):
        nxt = None
        if j + 1 < NOV:
            if j - 1 >= 0:
                cp_out[j - 1].wait()
            nxt = pltpu.async_copy(ov_hbm.at[idx8_v.at[j + 1]],
                                   ovb[(j + 1) % 2], sem_i)
        cp_in.wait()
        cp_out[j] = pltpu.async_copy(ovb[j % 2],
                                     ov_out.at[pl.ds(base + j * OVC, OVC)],
                                     sem_o)
        cp_in = nxt
    cp_out[NOV - 2].wait()
    cp_out[NOV - 1].wait()


@jax.jit
def _sc_gather(x, movie_table, meta, ov_emb, genre_table, coll_table):
    x4 = x.reshape(NW, NCH, C)
    x8 = x.reshape(NW, NOV, OVC)
    mesh = plsc.VectorSubcoreMesh(core_axis_name="c", subcore_axis_name="s")
    f = pl.kernel(
        _sc_body,
        out_type=[
            jax.ShapeDtypeStruct((B, D_MOVIE), jnp.float32),
            jax.ShapeDtypeStruct((B, D_GENRE), jnp.float32),
            jax.ShapeDtypeStruct((B, D_COLL), jnp.float32),
            jax.ShapeDtypeStruct((B, D_OV), jnp.float32),
        ],
        mesh=mesh,
        scratch_types=[
            pltpu.VMEM((NCH, C), jnp.int32),
            pltpu.VMEM((NOV, OVC), jnp.int32),
            pltpu.VMEM((G + 1, C), jnp.int32),
            pltpu.VMEM((C, 16), jnp.int32),
            pltpu.VMEM((G, C, D_GENRE), jnp.float32),
            pltpu.VMEM((C, D_GENRE), jnp.float32),
            pltpu.VMEM((C, D_COLL), jnp.float32),
            pltpu.VMEM((C, D_MOVIE), jnp.float32),
            pltpu.VMEM((OVC, D_OV), jnp.float32),
            pltpu.VMEM((OVC, D_OV), jnp.float32),
            pltpu.SemaphoreType.DMA,
            pltpu.SemaphoreType.DMA,
            pltpu.SemaphoreType.DMA,
        ],
    )
    return f(x4, x8, movie_table, meta, ov_emb, genre_table, coll_table)


def _mlp_body(mv_ref, gs_ref, cl_ref, ov_ref, w1m_ref, w1g_ref, w1c_ref,
              w1o_ref, b1_ref, w2_ref, b2_ref, out_ref):
    h = jnp.dot(ov_ref[...], w1o_ref[...], preferred_element_type=jnp.float32)
    h = h + jnp.dot(mv_ref[...], w1m_ref[...],
                    preferred_element_type=jnp.float32)
    h = h + jnp.dot(gs_ref[...], w1g_ref[...],
                    preferred_element_type=jnp.float32)
    h = h + jnp.dot(cl_ref[...], w1c_ref[...],
                    preferred_element_type=jnp.float32)
    h = jnp.maximum(h + b1_ref[...], 0.0)
    out_ref[...] = jnp.dot(h, w2_ref[...],
                           preferred_element_type=jnp.float32) + b2_ref[...]


TB = 2048


@jax.jit
def _mlp(mv, gs, cl, ov, w1m, w1g, w1c, w1o, b1, w2, b2):
    grid = (B // TB,)
    bspec = lambda d: pl.BlockSpec((TB, d), lambda i: (i, 0))
    wspec = lambda r, c: pl.BlockSpec((r, c), lambda i: (0, 0))
    return pl.pallas_call(
        _mlp_body,
        grid=grid,
        in_specs=[
            bspec(D_MOVIE), bspec(D_GENRE), bspec(D_COLL), bspec(D_OV),
            wspec(D_MOVIE, HID), wspec(D_GENRE, HID), wspec(D_COLL, HID),
            wspec(D_OV, HID), wspec(1, HID), wspec(HID, RANK), wspec(1, RANK),
        ],
        out_specs=pl.BlockSpec((TB, RANK), lambda i: (i, 0)),
        out_shape=jax.ShapeDtypeStruct((B, RANK), jnp.float32),
    )(mv, gs, cl, ov, w1m, w1g, w1c, w1o, b1, w2, b2)


def kernel(x, movie_table, genres_map, collection_map, ov_emb,
           genre_table, coll_table, W1, b1, W2, b2):
    meta = jnp.concatenate(
        [genres_map,
         jnp.broadcast_to(collection_map[:, None], (collection_map.shape[0], 8))],
        axis=1)
    mv, gs, cl, ov = _sc_gather(x, movie_table, meta, ov_emb,
                                genre_table, coll_table)
    w1m = W1[:D_MOVIE]
    w1g = W1[D_MOVIE:D_MOVIE + D_GENRE] * (1.0 / G)
    w1c = W1[D_MOVIE + D_GENRE:D_MOVIE + D_GENRE + D_COLL]
    w1o = W1[D_MOVIE + D_GENRE + D_COLL:]
    return _mlp(mv, gs, cl, ov, w1m, w1g, w1c, w1o,
                b1.reshape(1, HID), W2, b2.reshape(1, RANK))


# speedup vs baseline: 1.4675x; 1.4675x over previous
import functools

import jax
import jax.numpy as jnp
from jax import lax
from jax.experimental import pallas as pl
from jax.experimental.pallas import tpu as pltpu
from jax.experimental.pallas import tpu_sc as plsc

B = 16384
D_MOVIE = 64
G = 8
D_GENRE = 16
D_COLL = 16
D_OV = 768
HID = 64
RANK = 64

NC = 2
NS = 16
NW = NC * NS
S = B // NW
C = 128
NCH = S // C
OVC = 32
NOV = S // OVC


def _sc_body(x_hbm, movie_hbm, gcol_hbm, cmap_hbm, ov_hbm, gtab_hbm,
             ctab_hbm,
             movie_out, gsum_out, coll_out, ov_out,
             idx_v, col_v, grow_v, gacc_v, crow_v, mrow_v,
             ovb0, ovb1, sem_m, sem_ids, sem_tab,
             sem_i0, sem_i1, sem_o0, sem_o1):
    wid = lax.axis_index("s") * NC + lax.axis_index("c")
    base = wid * S

    pltpu.sync_copy(x_hbm.at[wid], idx_v)

    ovb = [ovb0, ovb1]
    sem_i = [sem_i0, sem_i1]
    sem_o = [sem_o0, sem_o1]

    def ov_gather(j, b):
        return pltpu.async_copy(ov_hbm.at[idx_v.at[pl.ds(j * OVC, OVC)]],
                                ovb[b], sem_i[b])

    cps_in = [ov_gather(0, 0), None]

    for j in range(NCH):
        ids = idx_v.at[pl.ds(j * C, C)]
        rbase = base + j * C
        cp_m = pltpu.async_copy(movie_hbm.at[ids], mrow_v, sem_m)
        cps_i = [pltpu.async_copy(gcol_hbm.at[k].at[ids], col_v.at[k],
                                  sem_ids)
                 for k in range(G)]
        cps_i.append(pltpu.async_copy(cmap_hbm.at[ids], col_v.at[G],
                                      sem_ids))
        for cp in cps_i:
            cp.wait()
        cps = [pltpu.async_copy(gtab_hbm.at[col_v.at[k]], grow_v.at[k],
                                sem_tab)
               for k in range(G)]
        cps.append(pltpu.async_copy(ctab_hbm.at[col_v.at[G]], crow_v,
                                    sem_tab))
        for cp in cps:
            cp.wait()
        for r in range(C):
            acc = grow_v[0, r, :]
            for k in range(1, G):
                acc = acc + grow_v[k, r, :]
            gacc_v[r, :] = acc
        pltpu.sync_copy(gacc_v, gsum_out.at[pl.ds(rbase, C)])
        pltpu.sync_copy(crow_v, coll_out.at[pl.ds(rbase, C)])
        cp_m.wait()
        pltpu.sync_copy(mrow_v, movie_out.at[pl.ds(rbase, C)])

    cp_out = [None, None]
    for j in range(NOV):
        b = j % 2
        if j + 1 < NOV:
            nb = (j + 1) % 2
            if cp_out[nb] is not None:
                cp_out[nb].wait()
            cps_in[nb] = ov_gather(j + 1, nb)
        cps_in[b].wait()
        cp_out[b] = pltpu.async_copy(ovb[b],
                                     ov_out.at[pl.ds(base + j * OVC, OVC)],
                                     sem_o[b])
    cp_out[0].wait()
    cp_out[1].wait()


@jax.jit
def _sc_gather(x, movie_table, gmapT, cmap, ov_emb, genre_table, coll_table):
    x2 = x.reshape(NW, S)
    mesh = plsc.VectorSubcoreMesh(core_axis_name="c", subcore_axis_name="s")
    f = pl.kernel(
        _sc_body,
        out_type=[
            jax.ShapeDtypeStruct((B, D_MOVIE), jnp.float32),
            jax.ShapeDtypeStruct((B, D_GENRE), jnp.float32),
            jax.ShapeDtypeStruct((B, D_COLL), jnp.float32),
            jax.ShapeDtypeStruct((B, D_OV), jnp.float32),
        ],
        mesh=mesh,
        compiler_params=pltpu.CompilerParams(use_tc_tiling_on_sc=False),
        scratch_types=[
            pltpu.VMEM((S,), jnp.int32),
            pltpu.VMEM((G + 1, C), jnp.int32),
            pltpu.VMEM((G, C, D_GENRE), jnp.float32),
            pltpu.VMEM((C, D_GENRE), jnp.float32),
            pltpu.VMEM((C, D_COLL), jnp.float32),
            pltpu.VMEM((C, D_MOVIE), jnp.float32),
            pltpu.VMEM((OVC, D_OV), jnp.float32),
            pltpu.VMEM((OVC, D_OV), jnp.float32),
            pltpu.SemaphoreType.DMA,
            pltpu.SemaphoreType.DMA,
            pltpu.SemaphoreType.DMA,
            pltpu.SemaphoreType.DMA,
            pltpu.SemaphoreType.DMA,
            pltpu.SemaphoreType.DMA,
            pltpu.SemaphoreType.DMA,
        ],
    )
    return f(x2, movie_table, gmapT, cmap, ov_emb, genre_table, coll_table)


def _mlp_body(mv_ref, gs_ref, cl_ref, ov_ref, w1m_ref, w1g_ref, w1c_ref,
              w1o_ref, b1_ref, w2_ref, b2_ref, out_ref):
    h = jnp.dot(ov_ref[...], w1o_ref[...], preferred_element_type=jnp.float32)
    h = h + jnp.dot(mv_ref[...], w1m_ref[...],
                    preferred_element_type=jnp.float32)
    h = h + jnp.dot(gs_ref[...], w1g_ref[...],
                    preferred_element_type=jnp.float32)
    h = h + jnp.dot(cl_ref[...], w1c_ref[...],
                    preferred_element_type=jnp.float32)
    h = jnp.maximum(h + b1_ref[...], 0.0)
    out_ref[...] = jnp.dot(h, w2_ref[...],
                           preferred_element_type=jnp.float32) + b2_ref[...]


TB = 2048


@jax.jit
def _mlp(mv, gs, cl, ov, w1m, w1g, w1c, w1o, b1, w2, b2):
    grid = (B // TB,)
    bspec = lambda d: pl.BlockSpec((TB, d), lambda i: (i, 0))
    wspec = lambda r, c: pl.BlockSpec((r, c), lambda i: (0, 0))
    return pl.pallas_call(
        _mlp_body,
        grid=grid,
        in_specs=[
            bspec(D_MOVIE), bspec(D_GENRE), bspec(D_COLL), bspec(D_OV),
            wspec(D_MOVIE, HID), wspec(D_GENRE, HID), wspec(D_COLL, HID),
            wspec(D_OV, HID), wspec(1, HID), wspec(HID, RANK), wspec(1, RANK),
        ],
        out_specs=pl.BlockSpec((TB, RANK), lambda i: (i, 0)),
        out_shape=jax.ShapeDtypeStruct((B, RANK), jnp.float32),
    )(mv, gs, cl, ov, w1m, w1g, w1c, w1o, b1, w2, b2)


def kernel(x, movie_table, genres_map, collection_map, ov_emb,
           genre_table, coll_table, W1, b1, W2, b2):
    gmapT = genres_map.T
    mv, gs, cl, ov = _sc_gather(x, movie_table, gmapT, collection_map,
                                ov_emb, genre_table, coll_table)
    w1m = W1[:D_MOVIE]
    w1g = W1[D_MOVIE:D_MOVIE + D_GENRE] * (1.0 / G)
    w1c = W1[D_MOVIE + D_GENRE:D_MOVIE + D_GENRE + D_COLL]
    w1o = W1[D_MOVIE + D_GENRE + D_COLL:]
    return _mlp(mv, gs, cl, ov, w1m, w1g, w1c, w1o,
                b1.reshape(1, HID), W2, b2.reshape(1, RANK))


# speedup vs baseline: 1.4707x; 1.0021x over previous
import functools

import jax
import jax.numpy as jnp
from jax import lax
from jax.experimental import pallas as pl
from jax.experimental.pallas import tpu as pltpu
from jax.experimental.pallas import tpu_sc as plsc

B = 16384
D_MOVIE = 64
G = 8
D_GENRE = 16
D_COLL = 16
D_OV = 768
HID = 64
RANK = 64

NC = 2
NS = 16
NW = NC * NS
S = B // NW
C = 128
NCH = S // C
OVR = 8
NBUF = 6
LOOK = 4
NOV = S // OVR


def _sc_body(x_hbm, movie_hbm, gcol_hbm, cmap_hbm, ov_hbm, gtab_hbm,
             ctab_hbm,
             movie_out, gsum_out, coll_out, ov_out,
             idx_v, col_v, grow_v, gacc_v, crow_v, mrow_v, ovb_v,
             sem_mv, sem_id0, sem_id1, sem_tab, sem_out,
             sem_i, sem_o):
    wid = lax.axis_index("s") * NC + lax.axis_index("c")
    base = wid * S

    pltpu.sync_copy(x_hbm.at[wid], idx_v)

    finals = []

    cps_mv = [pltpu.async_copy(movie_hbm.at[idx_v.at[pl.ds(c * C, C)]],
                               mrow_v.at[pl.ds(c * C, C)], sem_mv)
              for c in range(NCH)]

    def ov_gather(j):
        return pltpu.async_copy(
            ov_hbm.at[idx_v.at[pl.ds(j * OVR, OVR)]], ovb_v.at[j % NBUF],
            sem_i.at[j % NBUF])

    cpi = [None] * NBUF
    cpo = [None] * NBUF
    ov_next = [0]
    for j in range(LOOK):
        cpi[j % NBUF] = ov_gather(j)

    def ov_pump(n):
        for _ in range(n):
            j = ov_next[0]
            if j >= NOV:
                return
            b = j % NBUF
            cpi[b].wait()
            cpo[b] = pltpu.async_copy(
                ovb_v.at[b], ov_out.at[pl.ds(base + j * OVR, OVR)],
                sem_o.at[b])
            t = j + LOOK
            if t < NOV:
                bt = t % NBUF
                if cpo[bt] is not None:
                    cpo[bt].wait()
                cpi[bt] = ov_gather(t)
            ov_next[0] = j + 1

    sem_id = [sem_id0, sem_id1]

    def issue_ids(c):
        ids = idx_v.at[pl.ds(c * C, C)]
        cps = [pltpu.async_copy(gcol_hbm.at[k].at[ids],
                                col_v.at[k, pl.ds(c * C, C)], sem_id[c % 2])
               for k in range(G)]
        cps.append(pltpu.async_copy(cmap_hbm.at[ids],
                                    col_v.at[G, pl.ds(c * C, C)],
                                    sem_id[c % 2]))
        return cps

    pend_ids = issue_ids(0)
    for c in range(NCH):
        nxt_ids = issue_ids(c + 1) if c + 1 < NCH else None
        ov_pump(4)
        for cp in pend_ids:
            cp.wait()
        pend_ids = nxt_ids
        cps = [pltpu.async_copy(gtab_hbm.at[col_v.at[k, pl.ds(c * C, C)]],
                                grow_v.at[k], sem_tab)
               for k in range(G)]
        cps.append(pltpu.async_copy(ctab_hbm.at[col_v.at[G, pl.ds(c * C, C)]],
                                    crow_v.at[c], sem_tab))
        ov_pump(4)
        for cp in cps:
            cp.wait()

        def red_body(r, _):
            acc = grow_v[0, r, :]
            for k in range(1, G):
                acc = acc + grow_v[k, r, :]
            gacc_v[c, r, :] = acc
            return 0

        lax.fori_loop(0, C, red_body, 0, unroll=4)
        finals.append(pltpu.async_copy(
            gacc_v.at[c], gsum_out.at[pl.ds(base + c * C, C)], sem_out))
        finals.append(pltpu.async_copy(
            crow_v.at[c], coll_out.at[pl.ds(base + c * C, C)], sem_out))
        ov_pump(4)

    for cp in cps_mv:
        cp.wait()
    finals.append(pltpu.async_copy(mrow_v, movie_out.at[pl.ds(base, S)],
                                   sem_out))
    ov_pump(NOV)
    for b in range(NBUF):
        if cpo[b] is not None:
            cpo[b].wait()
    for cp in finals:
        cp.wait()


@jax.jit
def _sc_gather(x, movie_table, gmapT, cmap, ov_emb, genre_table, coll_table):
    x2 = x.reshape(NW, S)
    mesh = plsc.VectorSubcoreMesh(core_axis_name="c", subcore_axis_name="s")
    f = pl.kernel(
        _sc_body,
        out_type=[
            jax.ShapeDtypeStruct((B, D_MOVIE), jnp.float32),
            jax.ShapeDtypeStruct((B, D_GENRE), jnp.float32),
            jax.ShapeDtypeStruct((B, D_COLL), jnp.float32),
            jax.ShapeDtypeStruct((B, D_OV), jnp.float32),
        ],
        mesh=mesh,
        compiler_params=pltpu.CompilerParams(use_tc_tiling_on_sc=False),
        scratch_types=[
            pltpu.VMEM((S,), jnp.int32),
            pltpu.VMEM((G + 1, S), jnp.int32),
            pltpu.VMEM((G, C, D_GENRE), jnp.float32),
            pltpu.VMEM((NCH, C, D_GENRE), jnp.float32),
            pltpu.VMEM((NCH, C, D_COLL), jnp.float32),
            pltpu.VMEM((S, D_MOVIE), jnp.float32),
            pltpu.VMEM((NBUF, OVR, D_OV), jnp.float32),
            pltpu.SemaphoreType.DMA,
            pltpu.SemaphoreType.DMA,
            pltpu.SemaphoreType.DMA,
            pltpu.SemaphoreType.DMA,
            pltpu.SemaphoreType.DMA,
            pltpu.SemaphoreType.DMA((NBUF,)),
            pltpu.SemaphoreType.DMA((NBUF,)),
        ],
    )
    return f(x2, movie_table, gmapT, cmap, ov_emb, genre_table, coll_table)


def _mlp_body(mv_ref, gs_ref, cl_ref, ov_ref, w1m_ref, w1g_ref, w1c_ref,
              w1o_ref, b1_ref, w2_ref, b2_ref, out_ref):
    h = jnp.dot(ov_ref[...], w1o_ref[...], preferred_element_type=jnp.float32)
    h = h + jnp.dot(mv_ref[...], w1m_ref[...],
                    preferred_element_type=jnp.float32)
    h = h + jnp.dot(gs_ref[...], w1g_ref[...],
                    preferred_element_type=jnp.float32)
    h = h + jnp.dot(cl_ref[...], w1c_ref[...],
                    preferred_element_type=jnp.float32)
    h = jnp.maximum(h + b1_ref[...], 0.0)
    out_ref[...] = jnp.dot(h, w2_ref[...],
                           preferred_element_type=jnp.float32) + b2_ref[...]


TB = 2048


@jax.jit
def _mlp(mv, gs, cl, ov, w1m, w1g, w1c, w1o, b1, w2, b2):
    grid = (B // TB,)
    bspec = lambda d: pl.BlockSpec((TB, d), lambda i: (i, 0))
    wspec = lambda r, c: pl.BlockSpec((r, c), lambda i: (0, 0))
    return pl.pallas_call(
        _mlp_body,
        grid=grid,
        in_specs=[
            bspec(D_MOVIE), bspec(D_GENRE), bspec(D_COLL), bspec(D_OV),
            wspec(D_MOVIE, HID), wspec(D_GENRE, HID), wspec(D_COLL, HID),
            wspec(D_OV, HID), wspec(1, HID), wspec(HID, RANK), wspec(1, RANK),
        ],
        out_specs=pl.BlockSpec((TB, RANK), lambda i: (i, 0)),
        out_shape=jax.ShapeDtypeStruct((B, RANK), jnp.float32),
    )(mv, gs, cl, ov, w1m, w1g, w1c, w1o, b1, w2, b2)


def kernel(x, movie_table, genres_map, collection_map, ov_emb,
           genre_table, coll_table, W1, b1, W2, b2):
    gmapT = genres_map.T
    mv, gs, cl, ov = _sc_gather(x, movie_table, gmapT, collection_map,
                                ov_emb, genre_table, coll_table)
    w1m = W1[:D_MOVIE]
    w1g = W1[D_MOVIE:D_MOVIE + D_GENRE] * (1.0 / G)
    w1c = W1[D_MOVIE + D_GENRE:D_MOVIE + D_GENRE + D_COLL]
    w1o = W1[D_MOVIE + D_GENRE + D_COLL:]
    return _mlp(mv, gs, cl, ov, w1m, w1g, w1c, w1o,
                b1.reshape(1, HID), W2, b2.reshape(1, RANK))


# speedup vs baseline: 1.8638x; 1.2673x over previous
import functools

import jax
import jax.numpy as jnp
from jax import lax
from jax.experimental import pallas as pl
from jax.experimental.pallas import tpu as pltpu
from jax.experimental.pallas import tpu_sc as plsc

B = 16384
D_MOVIE = 64
G = 8
D_GENRE = 16
D_COLL = 16
D_OV = 768
HID = 64
RANK = 64

NC = 2
NS = 16
NW = NC * NS
S = B // NW
C = 128
NCH = S // C
OVR = 8
NBUF = 8
LOOK = 6
NOV = S // OVR


def _sc_body(x_hbm, movie_hbm, meta_hbm, cmap_hbm, ov_hbm, gtab_hbm,
             ctab_hbm,
             movie_out, gsum_out, coll_out, ov_out,
             idx_v, meta_v, gtab_v, ccol_v, gacc_v, crow_v, mrow_v, ovb_v,
             sem_mv, sem_mt0, sem_mt1, sem_cm, sem_tab, sem_out,
             sem_i, sem_o):
    wid = lax.axis_index("s") * NC + lax.axis_index("c")
    base = wid * S

    pltpu.sync_copy(x_hbm.at[wid], idx_v)
    pltpu.sync_copy(gtab_hbm, gtab_v)

    finals = []

    cps_mv = [pltpu.async_copy(movie_hbm.at[idx_v.at[pl.ds(c * C, C)]],
                               mrow_v.at[pl.ds(c * C, C)], sem_mv)
              for c in range(NCH)]
    cps_cm = [pltpu.async_copy(cmap_hbm.at[idx_v.at[pl.ds(c * C, C)]],
                               ccol_v.at[pl.ds(c * C, C)], sem_cm)
              for c in range(NCH)]

    def ov_gather(j):
        return pltpu.async_copy(
            ov_hbm.at[idx_v.at[pl.ds(j * OVR, OVR)]], ovb_v.at[j % NBUF],
            sem_i.at[j % NBUF])

    cpi = [None] * NBUF
    cpo = [None] * NBUF
    ov_next = [0]
    for j in range(LOOK):
        cpi[j % NBUF] = ov_gather(j)

    def ov_pump(n):
        for _ in range(n):
            j = ov_next[0]
            if j >= NOV:
                return
            b = j % NBUF
            cpi[b].wait()
            cpo[b] = pltpu.async_copy(
                ovb_v.at[b], ov_out.at[pl.ds(base + j * OVR, OVR)],
                sem_o.at[b])
            t = j + LOOK
            if t < NOV:
                bt = t % NBUF
                if cpo[bt] is not None:
                    cpo[bt].wait()
                cpi[bt] = ov_gather(t)
            ov_next[0] = j + 1

    sem_mt = [sem_mt0, sem_mt1]

    def issue_meta(c):
        return pltpu.async_copy(meta_hbm.at[idx_v.at[pl.ds(c * C, C)]],
                                meta_v.at[c % 2], sem_mt[c % 2])

    pend_meta = issue_meta(0)
    cp_ctab = [None] * NCH
    for c in range(NCH):
        nxt_meta = issue_meta(c + 1) if c + 1 < NCH else None
        ov_pump(4)
        pend_meta.wait()
        pend_meta = nxt_meta
        mv = meta_v.at[c % 2]

        def gen_body(r, _):
            v = mv[r, :]
            acc = gtab_v[v[0], :]
            for k in range(1, G):
                acc = acc + gtab_v[v[k], :]
            gacc_v[c, r, :] = acc
            return 0

        lax.fori_loop(0, C, gen_body, 0, unroll=2)
        finals.append(pltpu.async_copy(
            gacc_v.at[c], gsum_out.at[pl.ds(base + c * C, C)], sem_out))
        if c == 0:
            for cp in cps_cm:
                cp.wait()
            cp_ctab = [pltpu.async_copy(
                ctab_hbm.at[ccol_v.at[pl.ds(i * C, C)]], crow_v.at[i],
                sem_tab) for i in range(NCH)]
        ov_pump(4)

    for c in range(NCH):
        cp_ctab[c].wait()
        finals.append(pltpu.async_copy(
            crow_v.at[c], coll_out.at[pl.ds(base + c * C, C)], sem_out))
    for cp in cps_mv:
        cp.wait()
    finals.append(pltpu.async_copy(mrow_v, movie_out.at[pl.ds(base, S)],
                                   sem_out))
    ov_pump(NOV)
    for b in range(NBUF):
        if cpo[b] is not None:
            cpo[b].wait()
    for cp in finals:
        cp.wait()


@jax.jit
def _sc_gather(x, movie_table, meta, cmap, ov_emb, genre_table, coll_table):
    x2 = x.reshape(NW, S)
    mesh = plsc.VectorSubcoreMesh(core_axis_name="c", subcore_axis_name="s")
    f = pl.kernel(
        _sc_body,
        out_type=[
            jax.ShapeDtypeStruct((B, D_MOVIE), jnp.float32),
            jax.ShapeDtypeStruct((B, D_GENRE), jnp.float32),
            jax.ShapeDtypeStruct((B, D_COLL), jnp.float32),
            jax.ShapeDtypeStruct((B, D_OV), jnp.float32),
        ],
        mesh=mesh,
        compiler_params=pltpu.CompilerParams(use_tc_tiling_on_sc=False),
        scratch_types=[
            pltpu.VMEM((S,), jnp.int32),
            pltpu.VMEM((2, C, 16), jnp.int32),
            pltpu.VMEM((20, D_GENRE), jnp.float32),
            pltpu.VMEM((S,), jnp.int32),
            pltpu.VMEM((NCH, C, D_GENRE), jnp.float32),
            pltpu.VMEM((NCH, C, D_COLL), jnp.float32),
            pltpu.VMEM((S, D_MOVIE), jnp.float32),
            pltpu.VMEM((NBUF, OVR, D_OV), jnp.float32),
            pltpu.SemaphoreType.DMA,
            pltpu.SemaphoreType.DMA,
            pltpu.SemaphoreType.DMA,
            pltpu.SemaphoreType.DMA,
            pltpu.SemaphoreType.DMA,
            pltpu.SemaphoreType.DMA,
            pltpu.SemaphoreType.DMA((NBUF,)),
            pltpu.SemaphoreType.DMA((NBUF,)),
        ],
    )
    return f(x2, movie_table, meta, cmap, ov_emb, genre_table, coll_table)


def _mlp_body(mv_ref, gs_ref, cl_ref, ov_ref, w1m_ref, w1g_ref, w1c_ref,
              w1o_ref, b1_ref, w2_ref, b2_ref, out_ref):
    h = jnp.dot(ov_ref[...], w1o_ref[...], preferred_element_type=jnp.float32)
    h = h + jnp.dot(mv_ref[...], w1m_ref[...],
                    preferred_element_type=jnp.float32)
    h = h + jnp.dot(gs_ref[...], w1g_ref[...],
                    preferred_element_type=jnp.float32)
    h = h + jnp.dot(cl_ref[...], w1c_ref[...],
                    preferred_element_type=jnp.float32)
    h = jnp.maximum(h + b1_ref[...], 0.0)
    out_ref[...] = jnp.dot(h, w2_ref[...],
                           preferred_element_type=jnp.float32) + b2_ref[...]


TB = 2048


@jax.jit
def _mlp(mv, gs, cl, ov, w1m, w1g, w1c, w1o, b1, w2, b2):
    grid = (B // TB,)
    bspec = lambda d: pl.BlockSpec((TB, d), lambda i: (i, 0))
    wspec = lambda r, c: pl.BlockSpec((r, c), lambda i: (0, 0))
    return pl.pallas_call(
        _mlp_body,
        grid=grid,
        in_specs=[
            bspec(D_MOVIE), bspec(D_GENRE), bspec(D_COLL), bspec(D_OV),
            wspec(D_MOVIE, HID), wspec(D_GENRE, HID), wspec(D_COLL, HID),
            wspec(D_OV, HID), wspec(1, HID), wspec(HID, RANK), wspec(1, RANK),
        ],
        out_specs=pl.BlockSpec((TB, RANK), lambda i: (i, 0)),
        out_shape=jax.ShapeDtypeStruct((B, RANK), jnp.float32),
    )(mv, gs, cl, ov, w1m, w1g, w1c, w1o, b1, w2, b2)


def kernel(x, movie_table, genres_map, collection_map, ov_emb,
           genre_table, coll_table, W1, b1, W2, b2):
    v = genres_map.shape[0]
    meta = jnp.concatenate(
        [genres_map, jnp.zeros((v, 16 - G), jnp.int32)], axis=1)
    mv, gs, cl, ov = _sc_gather(x, movie_table, meta, collection_map,
                                ov_emb, genre_table, coll_table)
    w1m = W1[:D_MOVIE]
    w1g = W1[D_MOVIE:D_MOVIE + D_GENRE] * (1.0 / G)
    w1c = W1[D_MOVIE + D_GENRE:D_MOVIE + D_GENRE + D_COLL]
    w1o = W1[D_MOVIE + D_GENRE + D_COLL:]
    return _mlp(mv, gs, cl, ov, w1m, w1g, w1c, w1o,
                b1.reshape(1, HID), W2, b2.reshape(1, RANK))


# speedup vs baseline: 4.4486x; 2.3869x over previous
import functools

import jax
import jax.numpy as jnp
from jax import lax
from jax.experimental import pallas as pl
from jax.experimental.pallas import tpu as pltpu
from jax.experimental.pallas import tpu_sc as plsc

B = 16384
D_MOVIE = 64
G = 8
D_GENRE = 16
D_COLL = 16
D_OV = 768
HID = 64
RANK = 64

NC = 2
NS = 16
NW = NC * NS
S = B // NW
C = 128
NCH = S // C
OVR = 8
NBUF = 8
LOOK = 6
NOV = S // OVR


def _sc_big_body(x_hbm, movp_hbm, ov_hbm,
                 movp_out, ov_out,
                 idx_v, idxp_v, mp_v, ovb_v,
                 sem_mv, sem_out, sem_i, sem_o):
    wid = lax.axis_index("s") * NC + lax.axis_index("c")
    base = wid * S

    pltpu.sync_copy(x_hbm.at[wid], idx_v)
    for i in range(S // 16):
        idxp_v[pl.ds(i * 16, 16)] = idx_v[pl.ds(i * 16, 16)] >> 1
    cps_mv = [pltpu.async_copy(movp_hbm.at[idxp_v.at[pl.ds(c * C, C)]],
                               mp_v.at[pl.ds(c * C, C)], sem_mv)
              for c in range(NCH)]

    def ov_gather(j):
        return pltpu.async_copy(
            ov_hbm.at[idx_v.at[pl.ds(j * OVR, OVR)]], ovb_v.at[j % NBUF],
            sem_i.at[j % NBUF])

    cpi = [None] * NBUF
    cpo = [None] * NBUF
    for j in range(LOOK):
        cpi[j % NBUF] = ov_gather(j)
    for j in range(NOV):
        b = j % NBUF
        cpi[b].wait()
        cpo[b] = pltpu.async_copy(
            ovb_v.at[b], ov_out.at[pl.ds(base + j * OVR, OVR)],
            sem_o.at[b])
        t = j + LOOK
        if t < NOV:
            bt = t % NBUF
            if cpo[bt] is not None:
                cpo[bt].wait()
            cpi[bt] = ov_gather(t)
    for cp in cps_mv:
        cp.wait()
    cp_mp = pltpu.async_copy(mp_v, movp_out.at[pl.ds(base, S)], sem_out)
    for b in range(NBUF):
        if cpo[b] is not None:
            cpo[b].wait()
    cp_mp.wait()


def _sc_small_body(x_hbm, meta_hbm, cmap_hbm, gtab_hbm, ctab_hbm,
                   gsum_out, coll_out,
                   idx_v, meta_v, gtab_v, ccol_v, gacc_v, crow_v,
                   sem_mt0, sem_mt1, sem_cm, sem_tab, sem_out):
    wid = lax.axis_index("s") * NC + lax.axis_index("c")
    base = wid * S

    pltpu.sync_copy(x_hbm.at[wid], idx_v)
    pltpu.sync_copy(gtab_hbm, gtab_v)

    finals = []
    cps_cm = [pltpu.async_copy(cmap_hbm.at[idx_v.at[pl.ds(c * C, C)]],
                               ccol_v.at[pl.ds(c * C, C)], sem_cm)
              for c in range(NCH)]

    sem_mt = [sem_mt0, sem_mt1]

    def issue_meta(c):
        return pltpu.async_copy(meta_hbm.at[idx_v.at[pl.ds(c * C, C)]],
                                meta_v.at[c % 2], sem_mt[c % 2])

    pend_meta = issue_meta(0)
    cp_ctab = [None] * NCH
    for c in range(NCH):
        nxt_meta = issue_meta(c + 1) if c + 1 < NCH else None
        pend_meta.wait()
        pend_meta = nxt_meta
        mv = meta_v.at[c % 2]
        if c == 0:
            for cp in cps_cm:
                cp.wait()
            cp_ctab = [pltpu.async_copy(
                ctab_hbm.at[ccol_v.at[pl.ds(i * C, C)]], crow_v.at[i],
                sem_tab) for i in range(NCH)]

        def gen_body(r, _):
            v = mv[r, :]
            acc = gtab_v[v[0], :]
            for k in range(1, G):
                acc = acc + gtab_v[v[k], :]
            gacc_v[c, r, :] = acc
            return 0

        lax.fori_loop(0, C, gen_body, 0, unroll=2)
        finals.append(pltpu.async_copy(
            gacc_v.at[c], gsum_out.at[pl.ds(base + c * C, C)], sem_out))

    for c in range(NCH):
        cp_ctab[c].wait()
        finals.append(pltpu.async_copy(
            crow_v.at[c], coll_out.at[pl.ds(base + c * C, C)], sem_out))
    for cp in finals:
        cp.wait()


@jax.jit
def _sc_gather(x, movie_table, meta, cmap, ov_emb, genre_table, coll_table):
    x2 = x.reshape(NW, S)
    vocab = movie_table.shape[0]
    movp = movie_table.reshape(vocab // 2, 2 * D_MOVIE)
    mesh = plsc.VectorSubcoreMesh(core_axis_name="c", subcore_axis_name="s")
    big = pl.kernel(
        _sc_big_body,
        out_type=[
            jax.ShapeDtypeStruct((B, 2 * D_MOVIE), jnp.float32),
            jax.ShapeDtypeStruct((B, D_OV), jnp.float32),
        ],
        mesh=mesh,
        scratch_types=[
            pltpu.VMEM((S,), jnp.int32),
            pltpu.VMEM((S,), jnp.int32),
            pltpu.VMEM((S, 2 * D_MOVIE), jnp.float32),
            pltpu.VMEM((NBUF, OVR, D_OV), jnp.float32),
            pltpu.SemaphoreType.DMA,
            pltpu.SemaphoreType.DMA,
            pltpu.SemaphoreType.DMA((NBUF,)),
            pltpu.SemaphoreType.DMA((NBUF,)),
        ],
    )
    small = pl.kernel(
        _sc_small_body,
        out_type=[
            jax.ShapeDtypeStruct((B, D_GENRE), jnp.float32),
            jax.ShapeDtypeStruct((B, D_COLL), jnp.float32),
        ],
        mesh=mesh,
        compiler_params=pltpu.CompilerParams(use_tc_tiling_on_sc=False),
        scratch_types=[
            pltpu.VMEM((S,), jnp.int32),
            pltpu.VMEM((2, C, 16), jnp.int32),
            pltpu.VMEM((20, D_GENRE), jnp.float32),
            pltpu.VMEM((S,), jnp.int32),
            pltpu.VMEM((NCH, C, D_GENRE), jnp.float32),
            pltpu.VMEM((NCH, C, D_COLL), jnp.float32),
            pltpu.SemaphoreType.DMA,
            pltpu.SemaphoreType.DMA,
            pltpu.SemaphoreType.DMA,
            pltpu.SemaphoreType.DMA,
            pltpu.SemaphoreType.DMA,
        ],
    )
    mp, ov = big(x2, movp, ov_emb)
    gs, cl = small(x2, meta, cmap, genre_table, coll_table)
    return mp, gs, cl, ov


def _mlp_body(mv_ref, par_ref, gs_ref, cl_ref, ov_ref, w1m_ref, w1g_ref,
              w1c_ref, w1o_ref, b1_ref, w2_ref, b2_ref, out_ref):
    h = jnp.dot(ov_ref[...], w1o_ref[...], preferred_element_type=jnp.float32)
    me = jnp.dot(mv_ref[:, :D_MOVIE], w1m_ref[...],
                 preferred_element_type=jnp.float32)
    mo = jnp.dot(mv_ref[:, D_MOVIE:], w1m_ref[...],
                 preferred_element_type=jnp.float32)
    h = h + me + par_ref[...] * (mo - me)
    h = h + jnp.dot(gs_ref[...], w1g_ref[...],
                    preferred_element_type=jnp.float32)
    h = h + jnp.dot(cl_ref[...], w1c_ref[...],
                    preferred_element_type=jnp.float32)
    h = jnp.maximum(h + b1_ref[...], 0.0)
    out_ref[...] = jnp.dot(h, w2_ref[...],
                           preferred_element_type=jnp.float32) + b2_ref[...]


TB = 2048


@jax.jit
def _mlp(mv, par, gs, cl, ov, w1m, w1g, w1c, w1o, b1, w2, b2):
    grid = (B // TB,)
    bspec = lambda d: pl.BlockSpec((TB, d), lambda i: (i, 0))
    wspec = lambda r, c: pl.BlockSpec((r, c), lambda i: (0, 0))
    return pl.pallas_call(
        _mlp_body,
        grid=grid,
        in_specs=[
            bspec(2 * D_MOVIE), bspec(1), bspec(D_GENRE), bspec(D_COLL),
            bspec(D_OV),
            wspec(D_MOVIE, HID), wspec(D_GENRE, HID), wspec(D_COLL, HID),
            wspec(D_OV, HID), wspec(1, HID), wspec(HID, RANK), wspec(1, RANK),
        ],
        out_specs=pl.BlockSpec((TB, RANK), lambda i: (i, 0)),
        out_shape=jax.ShapeDtypeStruct((B, RANK), jnp.float32),
    )(mv, par, gs, cl, ov, w1m, w1g, w1c, w1o, b1, w2, b2)


def kernel(x, movie_table, genres_map, collection_map, ov_emb,
           genre_table, coll_table, W1, b1, W2, b2):
    v = genres_map.shape[0]
    meta = jnp.concatenate(
        [genres_map, jnp.zeros((v, 16 - G), jnp.int32)], axis=1)
    mv, gs, cl, ov = _sc_gather(x, movie_table, meta, collection_map,
                                ov_emb, genre_table, coll_table)
    par = (x & 1).astype(jnp.float32).reshape(B, 1)
    w1m = W1[:D_MOVIE]
    w1g = W1[D_MOVIE:D_MOVIE + D_GENRE] * (1.0 / G)
    w1c = W1[D_MOVIE + D_GENRE:D_MOVIE + D_GENRE + D_COLL]
    w1o = W1[D_MOVIE + D_GENRE + D_COLL:]
    return _mlp(mv, par, gs, cl, ov, w1m, w1g, w1c, w1o,
                b1.reshape(1, HID), W2, b2.reshape(1, RANK))


# speedup vs baseline: 5.0116x; 1.1266x over previous
import functools

import jax
import jax.numpy as jnp
from jax import lax
from jax.experimental import pallas as pl
from jax.experimental.pallas import tpu as pltpu
from jax.experimental.pallas import tpu_sc as plsc

B = 16384
D_MOVIE = 64
G = 8
D_GENRE = 16
D_COLL = 16
D_OV = 768
HID = 64
RANK = 64

NC = 2
NS = 16
NW = NC * NS
S = B // NW
C = 128
NCH = S // C
OVR = 8
NBUF = 8
LOOK = 6
NOV = S // OVR


def _sc_big_body(x_hbm, movp_hbm, ov_hbm,
                 movp_out, ov_out,
                 idx_v, idxp_v, mp_v, ovb_v,
                 sem_mv, sem_out, sem_i, sem_o):
    wid = lax.axis_index("s") * NC + lax.axis_index("c")
    base = wid * S

    pltpu.sync_copy(x_hbm.at[wid], idx_v)
    for i in range(S // 16):
        idxp_v[pl.ds(i * 16, 16)] = idx_v[pl.ds(i * 16, 16)] >> 1
    cps_mv = [pltpu.async_copy(movp_hbm.at[idxp_v.at[pl.ds(c * C, C)]],
                               mp_v.at[pl.ds(c * C, C)], sem_mv)
              for c in range(NCH)]

    def ov_gather(j):
        return pltpu.async_copy(
            ov_hbm.at[idx_v.at[pl.ds(j * OVR, OVR)]], ovb_v.at[j % NBUF],
            sem_i.at[j % NBUF])

    cpi = [None] * NBUF
    cpo = [None] * NBUF
    for j in range(LOOK):
        cpi[j % NBUF] = ov_gather(j)
    for j in range(NOV):
        b = j % NBUF
        cpi[b].wait()
        cpo[b] = pltpu.async_copy(
            ovb_v.at[b], ov_out.at[pl.ds(base + j * OVR, OVR)],
            sem_o.at[b])
        t = j + LOOK
        if t < NOV:
            bt = t % NBUF
            if cpo[bt] is not None:
                cpo[bt].wait()
            cpi[bt] = ov_gather(t)
    for cp in cps_mv:
        cp.wait()
    cp_mp = pltpu.async_copy(mp_v, movp_out.at[pl.ds(base, S)], sem_out)
    for b in range(NBUF):
        if cpo[b] is not None:
            cpo[b].wait()
    cp_mp.wait()


def _sc_small_body(x_hbm, gpair_hbm, cmap_hbm, gtab_hbm, ctab_hbm,
                   gsum_out, coll_out,
                   idx_v, idxp_v, meta_v, gtab_v, ccol_v, gacc_v, crow_v,
                   sem_mt0, sem_mt1, sem_cm, sem_tab, sem_out):
    wid = lax.axis_index("s") * NC + lax.axis_index("c")
    base = wid * S

    pltpu.sync_copy(x_hbm.at[wid], idx_v)
    pltpu.sync_copy(gtab_hbm, gtab_v)
    for i in range(S // 16):
        idxp_v[pl.ds(i * 16, 16)] = idx_v[pl.ds(i * 16, 16)] >> 1

    finals = []
    cps_cm = [pltpu.async_copy(cmap_hbm.at[idx_v.at[pl.ds(c * C, C)]],
                               ccol_v.at[pl.ds(c * C, C)], sem_cm)
              for c in range(NCH)]

    sem_mt = [sem_mt0, sem_mt1]

    def issue_meta(c):
        return pltpu.async_copy(gpair_hbm.at[idxp_v.at[pl.ds(c * C, C)]],
                                meta_v.at[c % 2], sem_mt[c % 2])

    pend_meta = issue_meta(0)
    cp_ctab = [None] * NCH
    for c in range(NCH):
        nxt_meta = issue_meta(c + 1) if c + 1 < NCH else None
        pend_meta.wait()
        pend_meta = nxt_meta
        mv = meta_v.at[c % 2]
        if c == 0:
            for cp in cps_cm:
                cp.wait()
            cp_ctab = [pltpu.async_copy(
                ctab_hbm.at[ccol_v.at[pl.ds(i * C, C)]], crow_v.at[i],
                sem_tab) for i in range(NCH)]

        def gen_body(g, _):
            pv = idx_v[pl.ds(c * C + g * 16, 16)] & 1
            for rr in range(16):
                v = mv[g * 16 + rr, :]
                acc_a = gtab_v[v[0], :]
                acc_b = gtab_v[v[G], :]
                for k in range(1, G):
                    acc_a = acc_a + gtab_v[v[k], :]
                    acc_b = acc_b + gtab_v[v[G + k], :]
                sel = jnp.where(pv[rr] > 0, acc_b, acc_a)
                gacc_v[c, g * 16 + rr, :] = sel
            return 0

        lax.fori_loop(0, C // 16, gen_body, 0)
        finals.append(pltpu.async_copy(
            gacc_v.at[c], gsum_out.at[pl.ds(base + c * C, C)], sem_out))

    for c in range(NCH):
        cp_ctab[c].wait()
        finals.append(pltpu.async_copy(
            crow_v.at[c], coll_out.at[pl.ds(base + c * C, C)], sem_out))
    for cp in finals:
        cp.wait()


@jax.jit
def _sc_gather(x, movie_table, meta, cmap, ov_emb, genre_table, coll_table):
    x2 = x.reshape(NW, S)
    vocab = movie_table.shape[0]
    movp = movie_table.reshape(vocab // 2, 2 * D_MOVIE)
    mesh = plsc.VectorSubcoreMesh(core_axis_name="c", subcore_axis_name="s")
    big = pl.kernel(
        _sc_big_body,
        out_type=[
            jax.ShapeDtypeStruct((B, 2 * D_MOVIE), jnp.float32),
            jax.ShapeDtypeStruct((B, D_OV), jnp.float32),
        ],
        mesh=mesh,
        scratch_types=[
            pltpu.VMEM((S,), jnp.int32),
            pltpu.VMEM((S,), jnp.int32),
            pltpu.VMEM((S, 2 * D_MOVIE), jnp.float32),
            pltpu.VMEM((NBUF, OVR, D_OV), jnp.float32),
            pltpu.SemaphoreType.DMA,
            pltpu.SemaphoreType.DMA,
            pltpu.SemaphoreType.DMA((NBUF,)),
            pltpu.SemaphoreType.DMA((NBUF,)),
        ],
    )
    small = pl.kernel(
        _sc_small_body,
        out_type=[
            jax.ShapeDtypeStruct((B, D_GENRE), jnp.float32),
            jax.ShapeDtypeStruct((B, D_COLL), jnp.float32),
        ],
        mesh=mesh,
        compiler_params=pltpu.CompilerParams(use_tc_tiling_on_sc=False),
        scratch_types=[
            pltpu.VMEM((S,), jnp.int32),
            pltpu.VMEM((S,), jnp.int32),
            pltpu.VMEM((2, C, 16), jnp.int32),
            pltpu.VMEM((20, D_GENRE), jnp.float32),
            pltpu.VMEM((S,), jnp.int32),
            pltpu.VMEM((NCH, C, D_GENRE), jnp.float32),
            pltpu.VMEM((NCH, C, D_COLL), jnp.float32),
            pltpu.SemaphoreType.DMA,
            pltpu.SemaphoreType.DMA,
            pltpu.SemaphoreType.DMA,
            pltpu.SemaphoreType.DMA,
            pltpu.SemaphoreType.DMA,
        ],
    )
    mp, ov = big(x2, movp, ov_emb)
    gpair = meta.reshape(meta.shape[0] // 2, 2 * G)
    gs, cl = small(x2, gpair, cmap, genre_table, coll_table)
    return mp, gs, cl, ov


def _mlp_body(mv_ref, par_ref, gs_ref, cl_ref, ov_ref, w1m_ref, w1g_ref,
              w1c_ref, w1o_ref, b1_ref, w2_ref, b2_ref, out_ref):
    h = jnp.dot(ov_ref[...], w1o_ref[...], preferred_element_type=jnp.float32)
    me = jnp.dot(mv_ref[:, :D_MOVIE], w1m_ref[...],
                 preferred_element_type=jnp.float32)
    mo = jnp.dot(mv_ref[:, D_MOVIE:], w1m_ref[...],
                 preferred_element_type=jnp.float32)
    h = h + me + par_ref[...] * (mo - me)
    h = h + jnp.dot(gs_ref[...], w1g_ref[...],
                    preferred_element_type=jnp.float32)
    h = h + jnp.dot(cl_ref[...], w1c_ref[...],
                    preferred_element_type=jnp.float32)
    h = jnp.maximum(h + b1_ref[...], 0.0)
    out_ref[...] = jnp.dot(h, w2_ref[...],
                           preferred_element_type=jnp.float32) + b2_ref[...]


TB = 2048


@jax.jit
def _mlp(mv, par, gs, cl, ov, w1m, w1g, w1c, w1o, b1, w2, b2):
    grid = (B // TB,)
    bspec = lambda d: pl.BlockSpec((TB, d), lambda i: (i, 0))
    wspec = lambda r, c: pl.BlockSpec((r, c), lambda i: (0, 0))
    return pl.pallas_call(
        _mlp_body,
        grid=grid,
        in_specs=[
            bspec(2 * D_MOVIE), bspec(1), bspec(D_GENRE), bspec(D_COLL),
            bspec(D_OV),
            wspec(D_MOVIE, HID), wspec(D_GENRE, HID), wspec(D_COLL, HID),
            wspec(D_OV, HID), wspec(1, HID), wspec(HID, RANK), wspec(1, RANK),
        ],
        out_specs=pl.BlockSpec((TB, RANK), lambda i: (i, 0)),
        out_shape=jax.ShapeDtypeStruct((B, RANK), jnp.float32),
    )(mv, par, gs, cl, ov, w1m, w1g, w1c, w1o, b1, w2, b2)


def kernel(x, movie_table, genres_map, collection_map, ov_emb,
           genre_table, coll_table, W1, b1, W2, b2):
    mv, gs, cl, ov = _sc_gather(x, movie_table, genres_map, collection_map,
                                ov_emb, genre_table, coll_table)
    par = (x & 1).astype(jnp.float32).reshape(B, 1)
    w1m = W1[:D_MOVIE]
    w1g = W1[D_MOVIE:D_MOVIE + D_GENRE] * (1.0 / G)
    w1c = W1[D_MOVIE + D_GENRE:D_MOVIE + D_GENRE + D_COLL]
    w1o = W1[D_MOVIE + D_GENRE + D_COLL:]
    return _mlp(mv, par, gs, cl, ov, w1m, w1g, w1c, w1o,
                b1.reshape(1, HID), W2, b2.reshape(1, RANK))


# speedup vs baseline: 5.5210x; 1.1016x over previous
import functools

import jax
import jax.numpy as jnp
from jax import lax
from jax.experimental import pallas as pl
from jax.experimental.pallas import tpu as pltpu
from jax.experimental.pallas import tpu_sc as plsc

B = 16384
D_MOVIE = 64
G = 8
D_GENRE = 16
D_COLL = 16
D_OV = 768
HID = 64
RANK = 64

NC = 2
NS = 16
NW = NC * NS
S = B // NW
C = 128
NCH = S // C
OVR = 8
NBUF = 8
LOOK = 6
NOV = S // OVR


def _sc_big_body(x_hbm, ov_hbm,
                 ov_out,
                 idx_v, ovb_v,
                 sem_i, sem_o):
    wid = lax.axis_index("s") * NC + lax.axis_index("c")
    base = wid * S

    pltpu.sync_copy(x_hbm.at[wid], idx_v)

    def ov_gather(j):
        return pltpu.async_copy(
            ov_hbm.at[idx_v.at[pl.ds(j * OVR, OVR)]], ovb_v.at[j % NBUF],
            sem_i.at[j % NBUF])

    cpi = [None] * NBUF
    cpo = [None] * NBUF
    for j in range(LOOK):
        cpi[j % NBUF] = ov_gather(j)
    for j in range(NOV):
        b = j % NBUF
        cpi[b].wait()
        cpo[b] = pltpu.async_copy(
            ovb_v.at[b], ov_out.at[pl.ds(base + j * OVR, OVR)],
            sem_o.at[b])
        t = j + LOOK
        if t < NOV:
            bt = t % NBUF
            if cpo[bt] is not None:
                cpo[bt].wait()
            cpi[bt] = ov_gather(t)
    for b in range(NBUF):
        if cpo[b] is not None:
            cpo[b].wait()


def _sc_small_body(x_hbm, gpair_hbm, cmap_hbm, movie_hbm, gtab_hbm, ctab_hbm,
                   comb_out,
                   idx_v, idxp_v, meta_v, gtab_v, ccol_v, gacc_v, crow_v,
                   mrow_v, zpad_v,
                   sem_mt0, sem_mt1, sem_cm, sem_mv, sem_tab, sem_out):
    wid = lax.axis_index("s") * NC + lax.axis_index("c")
    base = wid * S

    pltpu.sync_copy(x_hbm.at[wid], idx_v)
    pltpu.sync_copy(gtab_hbm, gtab_v)
    for i in range(S // 16):
        idxp_v[pl.ds(i * 16, 16)] = idx_v[pl.ds(i * 16, 16)] >> 1

    finals = []
    cps_mv = [pltpu.async_copy(movie_hbm.at[idx_v.at[pl.ds(c * C, C)]],
                               mrow_v.at[pl.ds(c * C, C)], sem_mv)
              for c in range(NCH)]
    cps_cm = [pltpu.async_copy(cmap_hbm.at[idx_v.at[pl.ds(c * C, C)]],
                               ccol_v.at[pl.ds(c * C, C)], sem_cm)
              for c in range(NCH)]

    sem_mt = [sem_mt0, sem_mt1]

    def issue_meta(c):
        return pltpu.async_copy(gpair_hbm.at[idxp_v.at[pl.ds(c * C, C)]],
                                meta_v.at[c % 2], sem_mt[c % 2])

    zero16 = jnp.zeros((16,), jnp.float32)

    def z_body(r, _):
        zpad_v[r, pl.ds(0, 16)] = zero16
        zpad_v[r, pl.ds(16, 16)] = zero16
        return 0

    lax.fori_loop(0, C, z_body, 0, unroll=4)

    pend_meta = issue_meta(0)
    cp_ctab = [None] * NCH
    for c in range(NCH):
        nxt_meta = issue_meta(c + 1) if c + 1 < NCH else None
        pend_meta.wait()
        pend_meta = nxt_meta
        mv = meta_v.at[c % 2]
        if c == 0:
            for cp in cps_cm:
                cp.wait()
            cp_ctab = [pltpu.async_copy(
                ctab_hbm.at[ccol_v.at[pl.ds(i * C, C)]], crow_v.at[i],
                sem_tab) for i in range(NCH)]

        def gen_body(g, _):
            pv = idx_v[pl.ds(c * C + g * 16, 16)] & 1
            for rr in range(16):
                v = mv[g * 16 + rr, :]
                acc_a = gtab_v[v[0], :]
                acc_b = gtab_v[v[G], :]
                for k in range(1, G):
                    acc_a = acc_a + gtab_v[v[k], :]
                    acc_b = acc_b + gtab_v[v[G + k], :]
                gacc_v[c, g * 16 + rr, :] = jnp.where(pv[rr] > 0, acc_b,
                                                      acc_a)
            return 0

        lax.fori_loop(0, C // 16, gen_body, 0)
        rows = pl.ds(base + c * C, C)
        finals.append(pltpu.async_copy(
            gacc_v.at[c], comb_out.at[rows, pl.ds(0, D_GENRE)], sem_out))
        finals.append(pltpu.async_copy(
            zpad_v, comb_out.at[rows, pl.ds(96, 32)], sem_out))

    for c in range(NCH):
        cp_ctab[c].wait()
        finals.append(pltpu.async_copy(
            crow_v.at[c],
            comb_out.at[pl.ds(base + c * C, C), pl.ds(D_GENRE, D_COLL)],
            sem_out))
    for cp in cps_mv:
        cp.wait()
    finals.append(pltpu.async_copy(
        mrow_v, comb_out.at[pl.ds(base, S), pl.ds(32, D_MOVIE)], sem_out))
    for cp in finals:
        cp.wait()


@jax.jit
def _sc_gather(x, movie_table, gmap, cmap, ov_emb, genre_table, coll_table):
    x2 = x.reshape(NW, S)
    mesh = plsc.VectorSubcoreMesh(core_axis_name="c", subcore_axis_name="s")
    big = pl.kernel(
        _sc_big_body,
        out_type=[
            jax.ShapeDtypeStruct((B, D_OV), jnp.float32),
        ],
        mesh=mesh,
        scratch_types=[
            pltpu.VMEM((S,), jnp.int32),
            pltpu.VMEM((NBUF, OVR, D_OV), jnp.float32),
            pltpu.SemaphoreType.DMA((NBUF,)),
            pltpu.SemaphoreType.DMA((NBUF,)),
        ],
    )
    small = pl.kernel(
        _sc_small_body,
        out_type=[
            jax.ShapeDtypeStruct((B, 128), jnp.float32),
        ],
        mesh=mesh,
        compiler_params=pltpu.CompilerParams(use_tc_tiling_on_sc=False),
        scratch_types=[
            pltpu.VMEM((S,), jnp.int32),
            pltpu.VMEM((S,), jnp.int32),
            pltpu.VMEM((2, C, 16), jnp.int32),
            pltpu.VMEM((20, D_GENRE), jnp.float32),
            pltpu.VMEM((S,), jnp.int32),
            pltpu.VMEM((NCH, C, D_GENRE), jnp.float32),
            pltpu.VMEM((NCH, C, D_COLL), jnp.float32),
            pltpu.VMEM((S, D_MOVIE), jnp.float32),
            pltpu.VMEM((C, 32), jnp.float32),
            pltpu.SemaphoreType.DMA,
            pltpu.SemaphoreType.DMA,
            pltpu.SemaphoreType.DMA,
            pltpu.SemaphoreType.DMA,
            pltpu.SemaphoreType.DMA,
            pltpu.SemaphoreType.DMA,
        ],
    )
    (ov,) = big(x2, ov_emb)
    gpair = gmap.reshape(gmap.shape[0] // 2, 2 * G)
    (comb,) = small(x2, gpair, cmap, movie_table, genre_table, coll_table)
    return comb, ov


def _mlp_body(cb_ref, ov_ref, w1cb_ref, w1o_ref, b1_ref, w2_ref, b2_ref,
              out_ref):
    h = jnp.dot(ov_ref[...], w1o_ref[...], preferred_element_type=jnp.float32)
    h = h + jnp.dot(cb_ref[...], w1cb_ref[...],
                    preferred_element_type=jnp.float32)
    h = jnp.maximum(h + b1_ref[...], 0.0)
    out_ref[...] = jnp.dot(h, w2_ref[...],
                           preferred_element_type=jnp.float32) + b2_ref[...]


TB = 2048


@jax.jit
def _mlp(cb, ov, w1cb, w1o, b1, w2, b2):
    grid = (B // TB,)
    bspec = lambda d: pl.BlockSpec((TB, d), lambda i: (i, 0))
    wspec = lambda r, c: pl.BlockSpec((r, c), lambda i: (0, 0))
    return pl.pallas_call(
        _mlp_body,
        grid=grid,
        in_specs=[
            bspec(128), bspec(D_OV),
            wspec(128, HID), wspec(D_OV, HID), wspec(1, HID),
            wspec(HID, RANK), wspec(1, RANK),
        ],
        out_specs=pl.BlockSpec((TB, RANK), lambda i: (i, 0)),
        out_shape=jax.ShapeDtypeStruct((B, RANK), jnp.float32),
    )(cb, ov, w1cb, w1o, b1, w2, b2)


def kernel(x, movie_table, genres_map, collection_map, ov_emb,
           genre_table, coll_table, W1, b1, W2, b2):
    comb, ov = _sc_gather(x, movie_table, genres_map, collection_map,
                          ov_emb, genre_table, coll_table)
    w1g = W1[D_MOVIE:D_MOVIE + D_GENRE] * (1.0 / G)
    w1c = W1[D_MOVIE + D_GENRE:D_MOVIE + D_GENRE + D_COLL]
    w1m = W1[:D_MOVIE]
    w1o = W1[D_MOVIE + D_GENRE + D_COLL:]
    w1cb = jnp.concatenate(
        [w1g, w1c, w1m, jnp.zeros((32, HID), jnp.float32)], axis=0)
    return _mlp(comb, ov, w1cb, w1o,
                b1.reshape(1, HID), W2, b2.reshape(1, RANK))
